# baseline (speedup 1.0000x reference)
import jax
import jax.numpy as jnp
from jax import lax
from jax.experimental import pallas as pl
from jax.experimental.pallas import tpu as pltpu
from jax.experimental.pallas import tpu_sc as plsc

N = 10000
E = 320000
F = 128
NS = 16
L = 16
EB = 64
SPB = 8
BPE = SPB * EB
NBLK = 42
NB = NBLK * SPB
EPT = NB * EB
E_PAD = EPT * NS
N_PAD = 10240
RPT = N_PAD // NS
BLK = 512


def _d_body(x_ref, dum_ref, o_ref):
    o_ref[...] = jnp.dot(x_ref[...], dum_ref[...],
                         preferred_element_type=jnp.float32)


def _final_body(x_ref, b_acc_ref, s_acc_ref, cnt_ref, wd_ref, w1_ref, ws_ref,
                bias_ref, o_ref):
    acc = jnp.dot(b_acc_ref[...], wd_ref[...],
                  preferred_element_type=jnp.float32)
    acc = acc + jnp.dot(s_acc_ref[...], w1_ref[...],
                        preferred_element_type=jnp.float32)
    acc = acc + jnp.dot(x_ref[...], ws_ref[...],
                        preferred_element_type=jnp.float32)
    cnt = jnp.sum(cnt_ref[...], axis=0)[:, None]
    conv = acc / (cnt + 1.0) + bias_ref[...]
    o_ref[...] = x_ref[...] + jnp.maximum(conv, 0.0)


def _sc_edges(sdp_ref, x_ref, d_ref, c01_ref,
              b_out, s_out, cnt_out,
              acc, rows0, rows1, rows2, rows3, bulk0, bulk1, bulk2,
              dstb0, dstb1, dstb2, dstb3, wb, dc, c01v,
              gsem0, gsem1, gsem2, gsem3, ssem0, ssem1, ssem2, ssem3,
              bsem0, bsem1, bsem2):
    cid = lax.axis_index("c")
    wid = lax.axis_index("s")
    rows = (rows0, rows1, rows2, rows3)
    bulk = (bulk0, bulk1, bulk2)
    dstb = (dstb0, dstb1, dstb2, dstb3)
    gsem = (gsem0, gsem1, gsem2, gsem3)
    ssem = (ssem0, ssem1, ssem2, ssem3)
    bsem = (bsem0, bsem1, bsem2)
    dloc = dc
    cntloc = dc

    def _zrow(r, _):
        for t in range(F // L):
            rows0[r, pl.ds(t * L, L)] = jnp.zeros((L,), jnp.float32)
            rows1[r, pl.ds(t * L, L)] = jnp.zeros((L,), jnp.float32)
        return 0
    lax.fori_loop(0, EB, _zrow, 0)
    for k in range(RPT // (2 * EB)):
        pltpu.sync_copy(rows0, acc.at[pl.ds(wid * RPT + 2 * k * EB, EB)])
        pltpu.sync_copy(rows1, acc.at[pl.ds(wid * RPT + (2 * k + 1) * EB, EB)])

    @pl.when(cid == 0)
    def _():
        pltpu.sync_copy(d_ref, dloc.at[pl.ds(0, N)])

    @pl.when(cid == 1)
    def _():
        def _zcnt(r, _):
            cntloc[pl.ds(r * L, L)] = jnp.zeros((L,), jnp.float32)
            return 0
        lax.fori_loop(0, N_PAD // L, _zcnt, 0)

    pltpu.sync_copy(c01_ref, c01v)
    plsc.subcore_barrier()

    bulk_base = wid * (NBLK * 2 * BPE)

    def _bulk_src(j):
        return sdp_ref.at[pl.ds(bulk_base + j * (2 * BPE), 2 * BPE)]

    def _gstart(idx_ref, s):
        pltpu.async_copy(x_ref.at[idx_ref], rows[s], gsem[s])

    def _gwait(idx_ref, s):
        pltpu.make_async_copy(x_ref.at[idx_ref], rows[s], gsem[s]).wait()

    def _compute_sc0(jb, k, s):
        c01 = c01v[...]
        for g in range(EB // L):
            sv = bulk[jb][pl.ds(k * EB + g * L, L)]
            dv = bulk[jb][pl.ds(BPE + k * EB + g * L, L)]
            dsv = plsc.load_gather(dloc, [sv])
            ddv = plsc.load_gather(dloc, [dv])
            z = dsv - ddv + c01
            w = 1.0 / (1.0 + jnp.exp(-z))
            w = jnp.where(sv == dv, 0.0, w)
            wb[pl.ds(g * L, L)] = w
            dstb[s][pl.ds(g * L, L)] = dv

        def _scale(j, _):
            wj = plsc.load_gather(wb, [jnp.full((L,), 0, jnp.int32) + j])
            for t in range(F // L):
                sl = pl.ds(t * L, L)
                rows[s][j, sl] = rows[s][j, sl] * wj
            return 0
        lax.fori_loop(0, EB, _scale, 0)

    def _compute_sc1(jb, k, s):
        ones = jnp.ones((L,), jnp.float32)
        for g in range(EB // L):
            sv = bulk[jb][pl.ds(k * EB + g * L, L)]
            dv = bulk[jb][pl.ds(BPE + k * EB + g * L, L)]
            valid = sv != dv
            dstb[s][pl.ds(g * L, L)] = jnp.where(valid, dv, N)
            plsc.addupdate_scatter(cntloc, [dv], ones, mask=valid)

    def _make_loop(compute):
        def _body(g, _):
            for jj in (0, 1, 2):
                j = 3 * g + jj
                nslot = (jj + 1) % 3
                pslot = (jj + 2) % 3

                @pl.when(j + 2 < NBLK)
                def _():
                    pltpu.async_copy(_bulk_src(j + 2), bulk[pslot],
                                     bsem[pslot])

                @pl.when(j + 1 < NBLK)
                def _():
                    pltpu.make_async_copy(_bulk_src(j + 1), bulk[nslot],
                                          bsem[nslot]).wait()
                for k in range(SPB):
                    i = j * SPB + k
                    s = k % 4
                    _gwait(bulk[jj].at[pl.ds(k * EB, EB)], s)
                    compute(jj, k, s)
                    pltpu.async_copy(rows[s], acc.at[dstb[s]], ssem[s],
                                     add=True)

                    @pl.when(i >= 2)
                    def _():
                        s2 = (s + 2) % 4
                        pltpu.make_async_copy(rows[s2], acc.at[dstb[s2]],
                                              ssem[s2]).wait()

                    @pl.when(i + 2 < NB)
                    def _():
                        s2 = (s + 2) % 4
                        if k < SPB - 2:
                            _gstart(bulk[jj].at[pl.ds((k + 2) * EB, EB)], s2)
                        else:
                            _gstart(
                                bulk[nslot].at[pl.ds((k - SPB + 2) * EB, EB)],
                                s2)
            return 0
        return _body

    pltpu.sync_copy(_bulk_src(0), bulk0)
    pltpu.async_copy(_bulk_src(1), bulk1, bsem1)
    _gstart(bulk0.at[pl.ds(0, EB)], 0)
    _gstart(bulk0.at[pl.ds(EB, EB)], 1)

    @pl.when(cid == 0)
    def _():
        lax.fori_loop(0, NBLK // 3, _make_loop(_compute_sc0), 0)

    @pl.when(cid == 1)
    def _():
        lax.fori_loop(0, NBLK // 3, _make_loop(_compute_sc1), 0)

    for s in ((NB - 2) % 4, (NB - 1) % 4):
        pltpu.make_async_copy(rows[s], acc.at[dstb[s]], ssem[s]).wait()

    plsc.subcore_barrier()
    stripe = pl.ds(wid * RPT, RPT)

    @pl.when(cid == 0)
    def _():
        pltpu.sync_copy(acc.at[stripe], b_out.at[stripe])

    @pl.when(cid == 1)
    def _():
        pltpu.sync_copy(acc.at[stripe], s_out.at[stripe])
        pltpu.sync_copy(cntloc, cnt_out.at[wid])


def _run_sc(sdp, x, d, c01):
    mesh = plsc.VectorSubcoreMesh(core_axis_name="c", subcore_axis_name="s")
    return pl.kernel(
        _sc_edges,
        out_type=(jax.ShapeDtypeStruct((N_PAD, F), jnp.float32),
                  jax.ShapeDtypeStruct((N_PAD, F), jnp.float32),
                  jax.ShapeDtypeStruct((NS, N_PAD), jnp.float32)),
        mesh=mesh,
        compiler_params=pltpu.CompilerParams(needs_layout_passes=False),
        scratch_types=[
            pltpu.VMEM_SHARED((N_PAD, F), jnp.float32),
            pltpu.VMEM((EB, F), jnp.float32),
            pltpu.VMEM((EB, F), jnp.float32),
            pltpu.VMEM((EB, F), jnp.float32),
            pltpu.VMEM((EB, F), jnp.float32),
            pltpu.VMEM((2 * BPE,), jnp.int32),
            pltpu.VMEM((2 * BPE,), jnp.int32),
            pltpu.VMEM((2 * BPE,), jnp.int32),
            pltpu.VMEM((EB,), jnp.int32),
            pltpu.VMEM((EB,), jnp.int32),
            pltpu.VMEM((EB,), jnp.int32),
            pltpu.VMEM((EB,), jnp.int32),
            pltpu.VMEM((EB,), jnp.float32),
            pltpu.VMEM((N_PAD,), jnp.float32),
            pltpu.VMEM((L,), jnp.float32),
            pltpu.SemaphoreType.DMA,
            pltpu.SemaphoreType.DMA,
            pltpu.SemaphoreType.DMA,
            pltpu.SemaphoreType.DMA,
            pltpu.SemaphoreType.DMA,
            pltpu.SemaphoreType.DMA,
            pltpu.SemaphoreType.DMA,
            pltpu.SemaphoreType.DMA,
            pltpu.SemaphoreType.DMA,
            pltpu.SemaphoreType.DMA,
            pltpu.SemaphoreType.DMA,
        ],
    )(sdp, x, d, c01)


def kernel(x, edge_index, W, U, c, b):
    W0 = W[:F]
    W1 = W[F:]
    qs = jax.nn.softmax(c)
    wd_t = (W0 - W1).T
    w1_t = W1.T
    ws_t = (qs[0] * W0 + qs[1] * W1).T
    du = U[0] - U[1]
    dum = jnp.zeros((F, 128), jnp.float32).at[:, 0].set(du)
    c01 = jnp.full((L,), c[0] - c[1], jnp.float32)

    src_p = jnp.zeros((E_PAD,), jnp.int32).at[:E].set(edge_index[0])
    dst_p = jnp.zeros((E_PAD,), jnp.int32).at[:E].set(edge_index[1])
    sdp = jnp.stack([src_p.reshape(NS, NBLK, BPE),
                     dst_p.reshape(NS, NBLK, BPE)], axis=2).reshape(-1)

    dmat = pl.pallas_call(
        _d_body,
        grid=(N // 400,),
        in_specs=[pl.BlockSpec((400, F), lambda i: (i, 0)),
                  pl.BlockSpec((F, 128), lambda i: (0, 0))],
        out_specs=pl.BlockSpec((400, 128), lambda i: (i, 0)),
        out_shape=jax.ShapeDtypeStruct((N, 128), jnp.float32),
    )(x, dum)
    d = dmat[:, 0]

    b_acc, s_acc, cnt_parts = _run_sc(sdp, x, d, c01)

    x_pad = jnp.zeros((N_PAD, F), jnp.float32).at[:N].set(x)

    out = pl.pallas_call(
        _final_body,
        grid=(N_PAD // BLK,),
        in_specs=[pl.BlockSpec((BLK, F), lambda i: (i, 0)),
                  pl.BlockSpec((BLK, F), lambda i: (i, 0)),
                  pl.BlockSpec((BLK, F), lambda i: (i, 0)),
                  pl.BlockSpec((NS, BLK), lambda i: (0, i)),
                  pl.BlockSpec((F, F), lambda i: (0, 0)),
                  pl.BlockSpec((F, F), lambda i: (0, 0)),
                  pl.BlockSpec((F, F), lambda i: (0, 0)),
                  pl.BlockSpec((1, F), lambda i: (0, 0))],
        out_specs=pl.BlockSpec((BLK, F), lambda i: (i, 0)),
        out_shape=jax.ShapeDtypeStruct((N_PAD, F), jnp.float32),
    )(x_pad, b_acc, s_acc, cnt_parts, wd_t, w1_t, ws_t, b.reshape(1, F))
    return out[:N]


# speedup vs baseline: 2.9462x; 2.9462x over previous
import jax
import jax.numpy as jnp
from jax import lax
from jax.experimental import pallas as pl
from jax.experimental.pallas import tpu as pltpu
from jax.experimental.pallas import tpu_sc as plsc

N = 10000
E = 320000
F = 128
NS = 16
L = 16
EB = 128
NB = 158
E_PAD = NB * EB * NS
N_PAD = 10240
RPT = N_PAD // NS
BLK = 512


def _d_body(x_ref, dum_ref, o_ref):
    o_ref[...] = jnp.dot(x_ref[...], dum_ref[...],
                         preferred_element_type=jnp.float32)


def _final_body(x_ref, b_acc_ref, s_acc_ref, cnt_ref, wd_ref, w1_ref, ws_ref,
                bias_ref, o_ref):
    acc = jnp.dot(b_acc_ref[...], wd_ref[...],
                  preferred_element_type=jnp.float32)
    acc = acc + jnp.dot(s_acc_ref[...], w1_ref[...],
                        preferred_element_type=jnp.float32)
    acc = acc + jnp.dot(x_ref[...], ws_ref[...],
                        preferred_element_type=jnp.float32)
    cnt = jnp.sum(cnt_ref[...], axis=0)[:, None]
    conv = acc / (cnt + 1.0) + bias_ref[...]
    o_ref[...] = x_ref[...] + jnp.maximum(conv, 0.0)


def _sc_edges(src_ref, dst_ref, x_ref, d_ref, c01_ref,
              b_out, s_out, cnt_out,
              acc, rows0, rows1, srcb0, srcb1, dstb0, dstb1, wb, dc, c01v,
              gsem0, gsem1):
    cid = lax.axis_index("c")
    wid = lax.axis_index("s")
    rows = (rows0, rows1)
    srcb = (srcb0, srcb1)
    dstb = (dstb0, dstb1)
    gsem = (gsem0, gsem1)
    dloc = dc
    cntloc = dc

    def _zrow(r, _):
        for t in range(F // L):
            rows0[r, pl.ds(t * L, L)] = jnp.zeros((L,), jnp.float32)
        return 0
    lax.fori_loop(0, EB, _zrow, 0)
    for k in range(RPT // EB):
        pltpu.sync_copy(rows0, acc.at[pl.ds(wid * RPT + k * EB, EB)])

    @pl.when(cid == 0)
    def _():
        pltpu.sync_copy(d_ref, dloc.at[pl.ds(0, N)])

    @pl.when(cid == 1)
    def _():
        def _zcnt(r, _):
            cntloc[pl.ds(r * L, L)] = jnp.zeros((L,), jnp.float32)
            return 0
        lax.fori_loop(0, N_PAD // L, _zcnt, 0)

    pltpu.sync_copy(c01_ref, c01v)
    plsc.subcore_barrier()

    base = wid * (NB * EB)

    def _fetch(i, b):
        off = base + i * EB
        pltpu.sync_copy(src_ref.at[pl.ds(off, EB)], srcb[b])
        pltpu.sync_copy(dst_ref.at[pl.ds(off, EB)], dstb[b])
        pltpu.async_copy(x_ref.at[srcb[b]], rows[b], gsem[b])

    def _gwait(b):
        pltpu.make_async_copy(x_ref.at[srcb[b]], rows[b], gsem[b]).wait()

    def _compute_sc0(b):
        c01 = c01v[...]
        for g in range(EB // L):
            sv = srcb[b][pl.ds(g * L, L)]
            dv = dstb[b][pl.ds(g * L, L)]
            dsv = plsc.load_gather(dloc, [sv])
            ddv = plsc.load_gather(dloc, [dv])
            z = dsv - ddv + c01
            w = 1.0 / (1.0 + jnp.exp(-z))
            w = jnp.where(sv == dv, 0.0, w)
            wb[pl.ds(g * L, L)] = w

        def _scale(g, _):
            wv = wb[pl.ds(g * L, L)]
            for j in range(L):
                wj = wv[j]
                row = g * L + j
                for t in range(F // L):
                    sl = pl.ds(t * L, L)
                    rows[b][row, sl] = rows[b][row, sl] * wj
            return 0
        lax.fori_loop(0, EB // L, _scale, 0)

    def _compute_sc1(b):
        ones = jnp.ones((L,), jnp.float32)
        for g in range(EB // L):
            sl = pl.ds(g * L, L)
            sv = srcb[b][sl]
            dv = dstb[b][sl]
            valid = sv != dv
            dstb[b][sl] = jnp.where(valid, dv, N)
            plsc.addupdate_scatter(cntloc, [dv], ones, mask=valid)

    def _make_loop(compute):
        def _body(g, _):
            for b in (0, 1):
                i = 2 * g + b
                _gwait(b)

                @pl.when(i + 1 < NB)
                def _():
                    _fetch(i + 1, 1 - b)
                compute(b)
                pltpu.sync_copy(rows[b], acc.at[dstb[b]], add=True)
            return 0
        return _body

    _fetch(0, 0)

    @pl.when(cid == 0)
    def _():
        lax.fori_loop(0, NB // 2, _make_loop(_compute_sc0), 0)

    @pl.when(cid == 1)
    def _():
        lax.fori_loop(0, NB // 2, _make_loop(_compute_sc1), 0)

    plsc.subcore_barrier()
    stripe = pl.ds(wid * RPT, RPT)

    @pl.when(cid == 0)
    def _():
        pltpu.sync_copy(acc.at[stripe], b_out.at[stripe])

    @pl.when(cid == 1)
    def _():
        pltpu.sync_copy(acc.at[stripe], s_out.at[stripe])
        pltpu.sync_copy(cntloc, cnt_out.at[wid])


def _run_sc(src_p, dst_p, x, d, c01):
    mesh = plsc.VectorSubcoreMesh(core_axis_name="c", subcore_axis_name="s")
    return pl.kernel(
        _sc_edges,
        out_type=(jax.ShapeDtypeStruct((N_PAD, F), jnp.float32),
                  jax.ShapeDtypeStruct((N_PAD, F), jnp.float32),
                  jax.ShapeDtypeStruct((NS, N_PAD), jnp.float32)),
        mesh=mesh,
        compiler_params=pltpu.CompilerParams(needs_layout_passes=False),
        scratch_types=[
            pltpu.VMEM_SHARED((N_PAD, F), jnp.float32),
            pltpu.VMEM((EB, F), jnp.float32),
            pltpu.VMEM((EB, F), jnp.float32),
            pltpu.VMEM((EB,), jnp.int32),
            pltpu.VMEM((EB,), jnp.int32),
            pltpu.VMEM((EB,), jnp.int32),
            pltpu.VMEM((EB,), jnp.int32),
            pltpu.VMEM((EB,), jnp.float32),
            pltpu.VMEM((N_PAD,), jnp.float32),
            pltpu.VMEM((L,), jnp.float32),
            pltpu.SemaphoreType.DMA,
            pltpu.SemaphoreType.DMA,
        ],
    )(src_p, dst_p, x, d, c01)


def kernel(x, edge_index, W, U, c, b):
    W0 = W[:F]
    W1 = W[F:]
    qs = jax.nn.softmax(c)
    wd_t = (W0 - W1).T
    w1_t = W1.T
    ws_t = (qs[0] * W0 + qs[1] * W1).T
    du = U[0] - U[1]
    dum = jnp.zeros((F, 128), jnp.float32).at[:, 0].set(du)
    c01 = jnp.full((L,), c[0] - c[1], jnp.float32)

    src_p = jnp.zeros((E_PAD,), jnp.int32).at[:E].set(edge_index[0])
    dst_p = jnp.zeros((E_PAD,), jnp.int32).at[:E].set(edge_index[1])

    dmat = pl.pallas_call(
        _d_body,
        grid=(N // 400,),
        in_specs=[pl.BlockSpec((400, F), lambda i: (i, 0)),
                  pl.BlockSpec((F, 128), lambda i: (0, 0))],
        out_specs=pl.BlockSpec((400, 128), lambda i: (i, 0)),
        out_shape=jax.ShapeDtypeStruct((N, 128), jnp.float32),
    )(x, dum)
    d = dmat[:, 0]

    b_acc, s_acc, cnt_parts = _run_sc(src_p, dst_p, x, d, c01)

    x_pad = jnp.zeros((N_PAD, F), jnp.float32).at[:N].set(x)

    out = pl.pallas_call(
        _final_body,
        grid=(N_PAD // BLK,),
        in_specs=[pl.BlockSpec((BLK, F), lambda i: (i, 0)),
                  pl.BlockSpec((BLK, F), lambda i: (i, 0)),
                  pl.BlockSpec((BLK, F), lambda i: (i, 0)),
                  pl.BlockSpec((NS, BLK), lambda i: (0, i)),
                  pl.BlockSpec((F, F), lambda i: (0, 0)),
                  pl.BlockSpec((F, F), lambda i: (0, 0)),
                  pl.BlockSpec((F, F), lambda i: (0, 0)),
                  pl.BlockSpec((1, F), lambda i: (0, 0))],
        out_specs=pl.BlockSpec((BLK, F), lambda i: (i, 0)),
        out_shape=jax.ShapeDtypeStruct((N_PAD, F), jnp.float32),
    )(x_pad, b_acc, s_acc, cnt_parts, wd_t, w1_t, ws_t, b.reshape(1, F))
    return out[:N]
